# Initial kernel scaffold; baseline (speedup 1.0000x reference)
import jax
import jax.numpy as jnp
from jax.experimental import pallas as pl


def kernel(h, e, edge_index, snorm_n, snorm_e, params, masks):
    raise NotImplementedError("write your pallas kernel here")


# speedup vs baseline: 1.4166x; 1.4166x over previous
import jax
import jax.numpy as jnp
from jax import lax
from jax.experimental import pallas as pl
from jax.experimental.pallas import tpu as pltpu
from jax.experimental.pallas import tpu_sc as plsc

N = 10000
E = 320000
D = 128
NCLS = 10

_BN_N = 2000
_BN_E = 3200

_B = 80
_NSUB = 16
_EPW = E // _NSUB
_NHALF = N // 2
_ACC = _NHALF + 8
_SPT = 312
_ZR = 104


def _mm_call(x, w, b, bn):
    m = x.shape[0]
    k = w.shape[1]

    def body(x_ref, w_ref, b_ref, o_ref):
        acc = jnp.dot(x_ref[...], w_ref[...],
                      preferred_element_type=jnp.float32)
        o_ref[...] = acc + b_ref[...]

    return pl.pallas_call(
        body,
        grid=(m // bn,),
        in_specs=[
            pl.BlockSpec((bn, 128), lambda i: (i, 0)),
            pl.BlockSpec((128, k), lambda i: (0, 0)),
            pl.BlockSpec((1, k), lambda i: (0, 0)),
        ],
        out_specs=pl.BlockSpec((bn, k), lambda i: (i, 0)),
        out_shape=jax.ShapeDtypeStruct((m, k), jnp.float32),
    )(x, w, b)


def _mm4_call(x, w, b, bn):
    m = x.shape[0]

    def body(x_ref, w_ref, b_ref, o0, o1, o2, o3):
        acc = jnp.dot(x_ref[...], w_ref[...],
                      preferred_element_type=jnp.float32)
        acc = acc + b_ref[...]
        o0[...] = acc[:, 0:128]
        o1[...] = acc[:, 128:256]
        o2[...] = acc[:, 256:384]
        o3[...] = acc[:, 384:512]

    blk = pl.BlockSpec((bn, 128), lambda i: (i, 0))
    return pl.pallas_call(
        body,
        grid=(m // bn,),
        in_specs=[
            pl.BlockSpec((bn, 128), lambda i: (i, 0)),
            pl.BlockSpec((128, 512), lambda i: (0, 0)),
            pl.BlockSpec((1, 512), lambda i: (0, 0)),
        ],
        out_specs=[blk] * 4,
        out_shape=[jax.ShapeDtypeStruct((m, 128), jnp.float32)] * 4,
    )(x, w, b)


def _edge_body(ce_hbm, src_hbm, dst_hbm, dh_hbm, eh_hbm, bh_hbm,
               enew_hbm, num_hbm, den_hbm,
               sidx, didx, dloc, ce_v, dh_v, eh_v, bh_v, enew_v, msg_v,
               zbuf, sg_sem, acc):
    c = lax.axis_index("c")
    s = lax.axis_index("s")
    lo = c * _NHALF

    def zinit(i, carry):
        zbuf[i // 8, pl.ds((i % 8) * 16, 16)] = jnp.zeros((16,), jnp.float32)
        return carry

    lax.fori_loop(0, _ZR * 8, zinit, 0)

    def zero_acc():
        for r in range(_SPT // _ZR):
            pltpu.sync_copy(zbuf, acc.at[pl.ds(s * _SPT + r * _ZR, _ZR)])

        @pl.when(s == _NSUB - 1)
        def _():
            tail = _ACC - _NSUB * _SPT
            pltpu.sync_copy(zbuf.at[pl.ds(0, tail)],
                            acc.at[pl.ds(_NSUB * _SPT, tail)])

    def flush_acc(out_hbm):
        pltpu.sync_copy(acc.at[pl.ds(s * _SPT, _SPT)],
                        out_hbm.at[pl.ds(lo + s * _SPT, _SPT)])

        @pl.when(s == _NSUB - 1)
        def _():
            tail = _NHALF - _NSUB * _SPT
            pltpu.sync_copy(acc.at[pl.ds(_NSUB * _SPT, tail)],
                            out_hbm.at[pl.ds(lo + _NSUB * _SPT, tail)])

    def make_dloc():
        def adj(i, cy):
            sl = pl.ds(i * 16, 16)
            d = didx[sl]
            inr = (d >= lo) & (d < lo + _NHALF)
            dloc[sl] = jnp.where(inr, d - lo, _NHALF)
            return cy

        lax.fori_loop(0, _B // 16, adj, 0)

    zero_acc()
    plsc.subcore_barrier()

    def chunk1(k, carry):
        base = s * _EPW + k * _B
        pltpu.sync_copy(src_hbm.at[pl.ds(base, _B)], sidx)
        pltpu.sync_copy(dst_hbm.at[pl.ds(base, _B)], didx)
        pltpu.async_copy(dh_hbm.at[sidx], dh_v, sg_sem)
        pltpu.async_copy(eh_hbm.at[didx], eh_v, sg_sem)
        pltpu.async_copy(bh_hbm.at[sidx], bh_v, sg_sem)
        pltpu.sync_copy(ce_hbm.at[pl.ds(base, _B)], ce_v)
        make_dloc()
        pltpu.make_async_copy(dh_hbm.at[sidx], dh_v, sg_sem).wait()
        pltpu.make_async_copy(eh_hbm.at[didx], eh_v, sg_sem).wait()
        pltpu.make_async_copy(bh_hbm.at[sidx], bh_v, sg_sem).wait()

        def ew(r_, cy):
            for f in range(8):
                sl = pl.ds(f * 16, 16)
                x = ce_v[r_, sl] + dh_v[r_, sl] + eh_v[r_, sl]
                sg = 1.0 / (1.0 + jnp.exp(-x))
                enew_v[r_, sl] = x
                msg_v[r_, sl] = sg * bh_v[r_, sl]
            return cy

        lax.fori_loop(0, _B, ew, 0)
        pltpu.sync_copy(enew_v, enew_hbm.at[pl.ds(base, _B)])
        pltpu.sync_copy(msg_v, acc.at[dloc], add=True)
        return carry

    lax.fori_loop(0, _EPW // _B, chunk1, 0)
    plsc.subcore_barrier()
    flush_acc(num_hbm)
    zero_acc()
    plsc.subcore_barrier()

    def chunk2(k, carry):
        base = s * _EPW + k * _B
        pltpu.sync_copy(dst_hbm.at[pl.ds(base, _B)], didx)
        make_dloc()
        pltpu.sync_copy(enew_hbm.at[pl.ds(base, _B)], enew_v)

        def ew(i, cy):
            r_ = i // 8
            sl = pl.ds((i % 8) * 16, 16)
            msg_v[r_, sl] = 1.0 / (1.0 + jnp.exp(-enew_v[r_, sl]))
            return cy

        lax.fori_loop(0, _B * 8, ew, 0)
        pltpu.sync_copy(msg_v, acc.at[dloc], add=True)
        return carry

    lax.fori_loop(0, _EPW // _B, chunk2, 0)
    plsc.subcore_barrier()
    flush_acc(den_hbm)


def _edge_call(ce, src, dst, dh, eh, bh):
    f32 = jnp.float32
    mesh = plsc.VectorSubcoreMesh(core_axis_name="c", subcore_axis_name="s")
    kern = pl.kernel(
        _edge_body,
        out_type=[
            jax.ShapeDtypeStruct((E, 128), f32),
            jax.ShapeDtypeStruct((N, 128), f32),
            jax.ShapeDtypeStruct((N, 128), f32),
        ],
        scratch_types=[
            pltpu.VMEM((_B,), jnp.int32),
            pltpu.VMEM((_B,), jnp.int32),
            pltpu.VMEM((_B,), jnp.int32),
            pltpu.VMEM((_B, 128), f32),
            pltpu.VMEM((_B, 128), f32),
            pltpu.VMEM((_B, 128), f32),
            pltpu.VMEM((_B, 128), f32),
            pltpu.VMEM((_B, 128), f32),
            pltpu.VMEM((_B, 128), f32),
            pltpu.VMEM((_ZR, 128), f32),
            pltpu.SemaphoreType.DMA,
            pltpu.VMEM_SHARED((_ACC, 128), f32),
        ],
        mesh=mesh,
    )
    return kern(ce, src, dst, dh, eh, bh)


def _estats_call(enew, snorm_e):
    nb = E // _BN_E

    def body(y_ref, sn_ref, s_ref, q_ref):
        i = pl.program_id(0)
        y = y_ref[...] * sn_ref[...]

        @pl.when(i == 0)
        def _():
            s_ref[...] = jnp.zeros_like(s_ref)
            q_ref[...] = jnp.zeros_like(q_ref)

        s_ref[...] += jnp.sum(y, axis=0, keepdims=True)
        q_ref[...] += jnp.sum(y * y, axis=0, keepdims=True)

    return pl.pallas_call(
        body,
        grid=(nb,),
        in_specs=[
            pl.BlockSpec((_BN_E, 128), lambda i: (i, 0)),
            pl.BlockSpec((_BN_E, 1), lambda i: (i, 0)),
        ],
        out_specs=[pl.BlockSpec((1, 128), lambda i: (0, 0))] * 2,
        out_shape=[jax.ShapeDtypeStruct((1, 128), jnp.float32)] * 2,
    )(enew, snorm_e)


def _eapply_call(enew, snorm_e, e_in, s, q, gg, bb):
    nb = E // _BN_E

    def body(y_ref, sn_ref, ein_ref, s_ref, q_ref, g_ref, b_ref, o_ref):
        y = y_ref[...] * sn_ref[...]
        mu = s_ref[...] / E
        var = q_ref[...] / E - mu * mu
        xb = g_ref[...] * (y - mu) / jnp.sqrt(var + 1e-5) + b_ref[...]
        o_ref[...] = ein_ref[...] + jnp.maximum(xb, 0.0)

    return pl.pallas_call(
        body,
        grid=(nb,),
        in_specs=[
            pl.BlockSpec((_BN_E, 128), lambda i: (i, 0)),
            pl.BlockSpec((_BN_E, 1), lambda i: (i, 0)),
            pl.BlockSpec((_BN_E, 128), lambda i: (i, 0)),
            pl.BlockSpec((1, 128), lambda i: (0, 0)),
            pl.BlockSpec((1, 128), lambda i: (0, 0)),
            pl.BlockSpec((1, 128), lambda i: (0, 0)),
            pl.BlockSpec((1, 128), lambda i: (0, 0)),
        ],
        out_specs=pl.BlockSpec((_BN_E, 128), lambda i: (i, 0)),
        out_shape=jax.ShapeDtypeStruct((E, 128), jnp.float32),
    )(enew, snorm_e, e_in, s, q, gg, bb)


def _hnew_call(ah, num, den, snorm_n):
    nb = N // _BN_N

    def body(a_ref, n_ref, d_ref, sn_ref, hs_ref, s_ref, q_ref):
        i = pl.program_id(0)
        hs = (a_ref[...] + n_ref[...] / (d_ref[...] + 1e-6)) * sn_ref[...]
        hs_ref[...] = hs

        @pl.when(i == 0)
        def _():
            s_ref[...] = jnp.zeros_like(s_ref)
            q_ref[...] = jnp.zeros_like(q_ref)

        s_ref[...] += jnp.sum(hs, axis=0, keepdims=True)
        q_ref[...] += jnp.sum(hs * hs, axis=0, keepdims=True)

    blk = pl.BlockSpec((_BN_N, 128), lambda i: (i, 0))
    return pl.pallas_call(
        body,
        grid=(nb,),
        in_specs=[blk, blk, blk, pl.BlockSpec((_BN_N, 1), lambda i: (i, 0))],
        out_specs=[blk,
                   pl.BlockSpec((1, 128), lambda i: (0, 0)),
                   pl.BlockSpec((1, 128), lambda i: (0, 0))],
        out_shape=[jax.ShapeDtypeStruct((N, 128), jnp.float32),
                   jax.ShapeDtypeStruct((1, 128), jnp.float32),
                   jax.ShapeDtypeStruct((1, 128), jnp.float32)],
    )(ah, num, den, snorm_n)


def _happly_call(hs, h_in, s, q, gg, bb):
    nb = N // _BN_N

    def body(hs_ref, hin_ref, s_ref, q_ref, g_ref, b_ref, o_ref, cs_ref):
        i = pl.program_id(0)
        mu = s_ref[...] / N
        var = q_ref[...] / N - mu * mu
        xb = g_ref[...] * (hs_ref[...] - mu) / jnp.sqrt(var + 1e-5) + b_ref[...]
        out = hin_ref[...] + jnp.maximum(xb, 0.0)
        o_ref[...] = out

        @pl.when(i == 0)
        def _():
            cs_ref[...] = jnp.zeros_like(cs_ref)

        cs_ref[...] += jnp.sum(out, axis=0, keepdims=True)

    blk = pl.BlockSpec((_BN_N, 128), lambda i: (i, 0))
    one = pl.BlockSpec((1, 128), lambda i: (0, 0))
    return pl.pallas_call(
        body,
        grid=(nb,),
        in_specs=[blk, blk, one, one, one, one],
        out_specs=[blk, one],
        out_shape=[jax.ShapeDtypeStruct((N, 128), jnp.float32),
                   jax.ShapeDtypeStruct((1, 128), jnp.float32)],
    )(hs, h_in, s, q, gg, bb)


def _readout_call(hcolsum, w, b):
    def body(cs_ref, w_ref, b_ref, o_ref):
        hg = cs_ref[...] / N
        o_ref[...] = jnp.dot(hg, w_ref[...],
                             preferred_element_type=jnp.float32) + b_ref[...]

    return pl.pallas_call(
        body,
        out_shape=jax.ShapeDtypeStruct((1, NCLS), jnp.float32),
    )(hcolsum, w, b)


def kernel(h, e, edge_index, snorm_n, snorm_e, params, masks):
    src = edge_index[0]
    dst = edge_index[1]

    h = _mm_call(h, params["emb_h"]["W"] * masks["emb_h"],
                 params["emb_h"]["b"].reshape(1, 128), _BN_N)
    e = _mm_call(e, params["emb_e"]["W"] * masks["emb_e"],
                 params["emb_e"]["b"].reshape(1, 128), _BN_E)

    hcs = None
    for lp, lm in zip(params["layers"], masks["layers"]):
        w4 = jnp.concatenate([lp[k]["W"] * lm[k] for k in ("A", "B", "D", "E")],
                             axis=1)
        b4 = jnp.concatenate([lp[k]["b"] for k in ("A", "B", "D", "E")])
        ah, bht, dht, eht = _mm4_call(h, w4, b4.reshape(1, 512), _BN_N)

        ce = _mm_call(e, lp["C"]["W"] * lm["C"],
                      lp["C"]["b"].reshape(1, 128), _BN_E)

        enew, num, den = _edge_call(ce, src, dst, dht, eht, bht)

        es, eq = _estats_call(enew, snorm_e)
        e = _eapply_call(enew, snorm_e, e, es, eq,
                         lp["ge"].reshape(1, 128), lp["be"].reshape(1, 128))

        hs, hsum, hsq = _hnew_call(ah, num, den, snorm_n)
        h, hcs = _happly_call(hs, h, hsum, hsq,
                              lp["gh"].reshape(1, 128), lp["bh"].reshape(1, 128))

    return _readout_call(hcs, params["readout"]["W"],
                         params["readout"]["b"].reshape(1, NCLS))
